# baseline (speedup 1.0000x reference)
import functools

import jax
import jax.numpy as jnp
from jax import lax
from jax.experimental import pallas as pl
from jax.experimental.pallas import tpu as pltpu
from jax.experimental.pallas import tpu_sc as plsc

_N = 5000
_NPAD = 5120
_C = 256
_T = _NPAD // _C
_BIG = 1e9

_T0 = 7
_R0 = _T0 * _C

_L = 16
_NW = 32
_NCHUNK = _NPAD // _L
_ROWS_PER_W = (_NPAD - _R0) // _NW


def _tc_kernel(box1_ref, b2c_ref, tri_ref, sumia_ref, cov2_ref, cols_ref, rows_ref):
    b1x1 = box1_ref[0:1, 0:1]
    b1y1 = box1_ref[0:1, 1:2]
    b1x2 = box1_ref[0:1, 2:3]
    b1y2 = box1_ref[0:1, 3:4]

    x1c = b2c_ref[0:1, :]
    y1c = b2c_ref[1:2, :]
    x2c = b2c_ref[2:3, :]
    y2c = b2c_ref[3:4, :]

    ix1 = jnp.maximum(b1x1, x1c)
    iy1 = jnp.maximum(b1y1, y1c)
    ix2 = jnp.minimum(b1x2, x2c)
    iy2 = jnp.minimum(b1y2, y2c)
    ia = jnp.maximum(ix2 - ix1 + 1.0, 0.0) * jnp.maximum(iy2 - iy1 + 1.0, 0.0)
    valid = ia > 0.0
    cols_ref[0:1, :] = jnp.where(valid, ix1, _BIG)
    cols_ref[1:2, :] = jnp.where(valid, iy1, _BIG)
    cols_ref[2:3, :] = jnp.where(valid, ix2, -_BIG) + 1.0
    cols_ref[3:4, :] = jnp.where(valid, iy2, -_BIG) + 1.0

    b2area = (x2c - x1c + 1.0) * (y2c - y1c + 1.0)
    cov2_ref[...] = ia / b2area

    sumia_ref[...] = jnp.sum(ia).reshape(1, 1)
    sum_ia_front = jnp.sum(ia[:, :_R0])

    zvec = jnp.zeros((1, _C), jnp.float32)

    def outer(i, carry):
        a_up, a_diag = carry
        rs = pl.ds(i * _C, _C)
        rows_ref[0:_C, :] = jnp.broadcast_to(cols_ref[0:1, rs], (_C, _C)).T
        rows_ref[_C : 2 * _C, :] = jnp.broadcast_to(cols_ref[1:2, rs], (_C, _C)).T
        rows_ref[2 * _C : 3 * _C, :] = jnp.broadcast_to(cols_ref[2:3, rs], (_C, _C)).T
        rows_ref[3 * _C : 4 * _C, :] = jnp.broadcast_to(cols_ref[3:4, rs], (_C, _C)).T

        def inner(j, carry2):
            su, sd = carry2
            cs = pl.ds(j * _C, _C)
            x_ext = jnp.maximum(rows_ref[2 * _C : 3 * _C, :], cols_ref[2:3, cs]) - jnp.maximum(
                rows_ref[0:_C, :], cols_ref[0:1, cs]
            )
            y_ext = jnp.maximum(rows_ref[3 * _C : 4 * _C, :], cols_ref[3:4, cs]) - jnp.maximum(
                rows_ref[_C : 2 * _C, :], cols_ref[1:2, cs]
            )
            p = jnp.maximum(x_ext, 0.0) * jnp.maximum(y_ext, 0.0)
            t = jnp.sum(p, axis=0, keepdims=True)
            is_diag = j == i
            su = su + jnp.where(is_diag, zvec, t)
            sd = sd + jnp.where(is_diag, t, zvec)
            return su, sd

        return lax.fori_loop(i, _T, inner, (a_up, a_diag))

    a_up, a_diag = lax.fori_loop(0, _T0, outer, (zvec, zvec))
    s_up = jnp.sum(a_up)
    s_diag = jnp.sum(a_diag)

    tri_ref[...] = (s_up + (s_diag - sum_ia_front) * 0.5).reshape(1, 1)


def _sc_kernel(b1_hbm, b2f_hbm, out_hbm, b1_v, raw_v, cols_v, acc_v):
    wid = lax.axis_index("s") * 2 + lax.axis_index("c")
    pltpu.sync_copy(b1_hbm, b1_v)
    pltpu.sync_copy(b2f_hbm, raw_v)

    b1vec = b1_v[...]
    b1x1 = jnp.full((_L,), b1vec[0], jnp.float32)
    b1y1 = jnp.full((_L,), b1vec[1], jnp.float32)
    b1x2 = jnp.full((_L,), b1vec[2], jnp.float32)
    b1y2 = jnp.full((_L,), b1vec[3], jnp.float32)

    def mask_body(k, carry):
        o = k * _L
        x1 = raw_v[pl.ds(o, _L)]
        y1 = raw_v[pl.ds(_NPAD + o, _L)]
        x2 = raw_v[pl.ds(2 * _NPAD + o, _L)]
        y2 = raw_v[pl.ds(3 * _NPAD + o, _L)]
        ix1 = jnp.maximum(b1x1, x1)
        iy1 = jnp.maximum(b1y1, y1)
        ix2 = jnp.minimum(b1x2, x2)
        iy2 = jnp.minimum(b1y2, y2)
        ia = jnp.maximum(ix2 - ix1 + 1.0, 0.0) * jnp.maximum(iy2 - iy1 + 1.0, 0.0)
        v = ia > 0.0
        cols_v[pl.ds(o, _L)] = jnp.where(v, ix1, _BIG)
        cols_v[pl.ds(_NPAD + o, _L)] = jnp.where(v, iy1, _BIG)
        cols_v[pl.ds(2 * _NPAD + o, _L)] = jnp.where(v, ix2, -_BIG) + 1.0
        cols_v[pl.ds(3 * _NPAD + o, _L)] = jnp.where(v, iy2, -_BIG) + 1.0
        return carry

    lax.fori_loop(0, _NCHUNK, mask_body, 0)

    zero = jnp.zeros((_L,), jnp.float32)

    def chunk_sum(rx1, ry1, rx2p, ry2p, k):
        o = k * _L
        x_ext = jnp.maximum(rx2p, cols_v[pl.ds(2 * _NPAD + o, _L)]) - jnp.maximum(
            rx1, cols_v[pl.ds(o, _L)]
        )
        y_ext = jnp.maximum(ry2p, cols_v[pl.ds(3 * _NPAD + o, _L)]) - jnp.maximum(
            ry1, cols_v[pl.ds(_NPAD + o, _L)]
        )
        return jnp.maximum(x_ext, 0.0) * jnp.maximum(y_ext, 0.0)

    def row_body(m, acc):
        r = _R0 + wid + m * _NW
        k0 = r // _L
        ridx = jnp.full((_L,), r, jnp.int32)
        rx1 = plsc.load_gather(cols_v, [ridx])
        ry1 = plsc.load_gather(cols_v, [ridx + _NPAD])
        rx2p = plsc.load_gather(cols_v, [ridx + 2 * _NPAD])
        ry2p = plsc.load_gather(cols_v, [ridx + 3 * _NPAD])
        b_chunk = chunk_sum(rx1, ry1, rx2p, ry2p, k0)
        acc = acc + 0.5 * b_chunk

        def tail(k, a):
            return a + chunk_sum(rx1, ry1, rx2p, ry2p, k)

        return lax.fori_loop(k0 + 1, _NCHUNK, tail, acc)

    acc = lax.fori_loop(0, _ROWS_PER_W, row_body, zero)

    def ia_body(k, a):
        o = k * _L
        x1 = raw_v[pl.ds(o, _L)]
        y1 = raw_v[pl.ds(_NPAD + o, _L)]
        x2 = raw_v[pl.ds(2 * _NPAD + o, _L)]
        y2 = raw_v[pl.ds(3 * _NPAD + o, _L)]
        ia = jnp.maximum(
            jnp.minimum(b1x2, x2) - jnp.maximum(b1x1, x1) + 1.0, 0.0
        ) * jnp.maximum(jnp.minimum(b1y2, y2) - jnp.maximum(b1y1, y1) + 1.0, 0.0)
        return a - 0.5 * ia

    @pl.when(wid == 0)
    def _():
        acc_v[...] = lax.fori_loop(_R0 // _L, _NCHUNK, ia_body, acc)

    @pl.when(wid != 0)
    def _():
        acc_v[...] = acc

    pltpu.sync_copy(acc_v, out_hbm.at[wid])


def kernel(box1, box2):
    box1 = box1.astype(jnp.float32)
    pad = jnp.tile(
        jnp.array([[_BIG, _BIG, -_BIG, -_BIG]], dtype=jnp.float32),
        (_NPAD - _N, 1),
    )
    b2p = jnp.concatenate([box2.astype(jnp.float32), pad], axis=0)
    b2c = b2p.T

    tri_tc, sumia, cov2 = pl.pallas_call(
        _tc_kernel,
        out_shape=[
            jax.ShapeDtypeStruct((1, 1), jnp.float32),
            jax.ShapeDtypeStruct((1, 1), jnp.float32),
            jax.ShapeDtypeStruct((1, _NPAD), jnp.float32),
        ],
        scratch_shapes=[
            pltpu.VMEM((8, _NPAD), jnp.float32),
            pltpu.VMEM((4 * _C, _C), jnp.float32),
        ],
    )(box1, b2c)

    b1_flat = jnp.pad(box1.reshape(4), (0, 12))
    b2_flat = b2c.reshape(-1)

    sc_call = functools.partial(
        pl.kernel,
        mesh=plsc.VectorSubcoreMesh(core_axis_name="c", subcore_axis_name="s"),
        out_type=jax.ShapeDtypeStruct((_NW, _L), jnp.float32),
        compiler_params=pltpu.CompilerParams(needs_layout_passes=False),
        scratch_types=[
            pltpu.VMEM((_L,), jnp.float32),
            pltpu.VMEM((4 * _NPAD,), jnp.float32),
            pltpu.VMEM((4 * _NPAD,), jnp.float32),
            pltpu.VMEM((_L,), jnp.float32),
        ],
    )(_sc_kernel)
    sc_parts = sc_call(b1_flat, b2_flat)

    tri = tri_tc[0, 0] + jnp.sum(sc_parts)
    b1_area = (box1[:, 2] - box1[:, 0] + 1.0) * (box1[:, 3] - box1[:, 1] + 1.0)
    cov1 = (sumia[0, 0] - tri) / b1_area
    return cov1, cov2[0, :_N]


# speedup vs baseline: 1.4074x; 1.4074x over previous
import functools

import jax
import jax.numpy as jnp
from jax import lax
from jax.experimental import pallas as pl
from jax.experimental.pallas import tpu as pltpu
from jax.experimental.pallas import tpu_sc as plsc

_N = 5000
_NPAD = 5120
_C = 256
_T = _NPAD // _C
_BIG = 1e9

_T0 = 7
_R0 = _T0 * _C

_L = 16
_NW = 32
_NCHUNK = _NPAD // _L
_ROWS_PER_W = (_NPAD - _R0) // _NW


def _tc_kernel(box1_ref, b2c_ref, tri_ref, sumia_ref, cov2_ref, cols_ref, rows_ref):
    b1x1 = box1_ref[0:1, 0:1]
    b1y1 = box1_ref[0:1, 1:2]
    b1x2 = box1_ref[0:1, 2:3]
    b1y2 = box1_ref[0:1, 3:4]

    x1c = b2c_ref[0:1, :]
    y1c = b2c_ref[1:2, :]
    x2c = b2c_ref[2:3, :]
    y2c = b2c_ref[3:4, :]

    ix1 = jnp.maximum(b1x1, x1c)
    iy1 = jnp.maximum(b1y1, y1c)
    ix2 = jnp.minimum(b1x2, x2c)
    iy2 = jnp.minimum(b1y2, y2c)
    ia = jnp.maximum(ix2 - ix1 + 1.0, 0.0) * jnp.maximum(iy2 - iy1 + 1.0, 0.0)
    valid = ia > 0.0
    cols_ref[0:1, :] = jnp.where(valid, ix1, _BIG)
    cols_ref[1:2, :] = jnp.where(valid, iy1, _BIG)
    cols_ref[2:3, :] = jnp.where(valid, ix2, -_BIG) + 1.0
    cols_ref[3:4, :] = jnp.where(valid, iy2, -_BIG) + 1.0

    b2area = (x2c - x1c + 1.0) * (y2c - y1c + 1.0)
    cov2_ref[...] = ia / b2area

    sumia_ref[...] = jnp.sum(ia).reshape(1, 1)
    sum_ia_front = jnp.sum(ia[:, :_R0])

    zvec = jnp.zeros((1, _C), jnp.float32)

    def outer(i, carry):
        a_up, a_diag = carry
        rs = pl.ds(i * _C, _C)
        rows_ref[0:_C, :] = jnp.broadcast_to(cols_ref[0:1, rs], (_C, _C)).T
        rows_ref[_C : 2 * _C, :] = jnp.broadcast_to(cols_ref[1:2, rs], (_C, _C)).T
        rows_ref[2 * _C : 3 * _C, :] = jnp.broadcast_to(cols_ref[2:3, rs], (_C, _C)).T
        rows_ref[3 * _C : 4 * _C, :] = jnp.broadcast_to(cols_ref[3:4, rs], (_C, _C)).T

        def inner(j, carry2):
            su, sd = carry2
            cs = pl.ds(j * _C, _C)
            x_ext = jnp.maximum(rows_ref[2 * _C : 3 * _C, :], cols_ref[2:3, cs]) - jnp.maximum(
                rows_ref[0:_C, :], cols_ref[0:1, cs]
            )
            y_ext = jnp.maximum(rows_ref[3 * _C : 4 * _C, :], cols_ref[3:4, cs]) - jnp.maximum(
                rows_ref[_C : 2 * _C, :], cols_ref[1:2, cs]
            )
            p = jnp.maximum(x_ext, 0.0) * jnp.maximum(y_ext, 0.0)
            t = jnp.sum(p, axis=0, keepdims=True)
            is_diag = j == i
            su = su + jnp.where(is_diag, zvec, t)
            sd = sd + jnp.where(is_diag, t, zvec)
            return su, sd

        return lax.fori_loop(i, _T, inner, (a_up, a_diag))

    a_up, a_diag = lax.fori_loop(0, _T0, outer, (zvec, zvec))
    s_up = jnp.sum(a_up)
    s_diag = jnp.sum(a_diag)

    tri_ref[...] = (s_up + (s_diag - sum_ia_front) * 0.5).reshape(1, 1)


def _sc_kernel(b1_hbm, b2f_hbm, out_hbm, b1_v, raw_v, cols_v, acc_v):
    wid = lax.axis_index("s") * 2 + lax.axis_index("c")
    pltpu.sync_copy(b1_hbm, b1_v)
    pltpu.sync_copy(b2f_hbm, raw_v)

    b1vec = b1_v[...]
    b1x1 = jnp.full((_L,), b1vec[0], jnp.float32)
    b1y1 = jnp.full((_L,), b1vec[1], jnp.float32)
    b1x2 = jnp.full((_L,), b1vec[2], jnp.float32)
    b1y2 = jnp.full((_L,), b1vec[3], jnp.float32)

    def mask_body(k, carry):
        o = k * _L
        x1 = raw_v[pl.ds(o, _L)]
        y1 = raw_v[pl.ds(_NPAD + o, _L)]
        x2 = raw_v[pl.ds(2 * _NPAD + o, _L)]
        y2 = raw_v[pl.ds(3 * _NPAD + o, _L)]
        ix1 = jnp.maximum(b1x1, x1)
        iy1 = jnp.maximum(b1y1, y1)
        ix2 = jnp.minimum(b1x2, x2)
        iy2 = jnp.minimum(b1y2, y2)
        ia = jnp.maximum(ix2 - ix1 + 1.0, 0.0) * jnp.maximum(iy2 - iy1 + 1.0, 0.0)
        v = ia > 0.0
        cols_v[pl.ds(o, _L)] = jnp.where(v, ix1, _BIG)
        cols_v[pl.ds(_NPAD + o, _L)] = jnp.where(v, iy1, _BIG)
        cols_v[pl.ds(2 * _NPAD + o, _L)] = jnp.where(v, ix2, -_BIG) + 1.0
        cols_v[pl.ds(3 * _NPAD + o, _L)] = jnp.where(v, iy2, -_BIG) + 1.0
        return carry

    lax.fori_loop(0, _NCHUNK, mask_body, 0)

    zero = jnp.zeros((_L,), jnp.float32)

    def row_body(m, acc):
        base = _R0 + wid + m * (4 * _NW)
        rows = []
        for q in range(4):
            ridx = jnp.full((_L,), base + q * _NW, jnp.int32)
            rows.append(
                (
                    plsc.load_gather(cols_v, [ridx]),
                    plsc.load_gather(cols_v, [ridx + _NPAD]),
                    plsc.load_gather(cols_v, [ridx + 2 * _NPAD]),
                    plsc.load_gather(cols_v, [ridx + 3 * _NPAD]),
                )
            )

        def chunk_sum4(k):
            o = k * _L
            cx1 = cols_v[pl.ds(o, _L)]
            cy1 = cols_v[pl.ds(_NPAD + o, _L)]
            cx2p = cols_v[pl.ds(2 * _NPAD + o, _L)]
            cy2p = cols_v[pl.ds(3 * _NPAD + o, _L)]
            t = None
            for rx1, ry1, rx2p, ry2p in rows:
                x_ext = jnp.maximum(rx2p, cx2p) - jnp.maximum(rx1, cx1)
                y_ext = jnp.maximum(ry2p, cy2p) - jnp.maximum(ry1, cy1)
                p = jnp.maximum(x_ext, 0.0) * jnp.maximum(y_ext, 0.0)
                t = p if t is None else t + p
            return t

        kb0 = (base // (8 * _L)) * 8
        for q in range(8):
            acc = acc + 0.5 * chunk_sum4(kb0 + q)

        def tail(k, a):
            return a + chunk_sum4(k)

        return lax.fori_loop(kb0 + 8, _NCHUNK, tail, acc)

    acc = lax.fori_loop(0, _ROWS_PER_W // 4, row_body, zero)

    def ia_body(k, a):
        o = k * _L
        x1 = raw_v[pl.ds(o, _L)]
        y1 = raw_v[pl.ds(_NPAD + o, _L)]
        x2 = raw_v[pl.ds(2 * _NPAD + o, _L)]
        y2 = raw_v[pl.ds(3 * _NPAD + o, _L)]
        ia = jnp.maximum(
            jnp.minimum(b1x2, x2) - jnp.maximum(b1x1, x1) + 1.0, 0.0
        ) * jnp.maximum(jnp.minimum(b1y2, y2) - jnp.maximum(b1y1, y1) + 1.0, 0.0)
        return a - 0.5 * ia

    @pl.when(wid == 0)
    def _():
        acc_v[...] = lax.fori_loop(_R0 // _L, _NCHUNK, ia_body, acc)

    @pl.when(wid != 0)
    def _():
        acc_v[...] = acc

    pltpu.sync_copy(acc_v, out_hbm.at[wid])


def kernel(box1, box2):
    box1 = box1.astype(jnp.float32)
    pad = jnp.tile(
        jnp.array([[_BIG, _BIG, -_BIG, -_BIG]], dtype=jnp.float32),
        (_NPAD - _N, 1),
    )
    b2p = jnp.concatenate([box2.astype(jnp.float32), pad], axis=0)
    b2c = b2p.T

    b1_flat = jnp.pad(box1.reshape(4), (0, 12))
    b2_flat = b2c.reshape(-1)

    sc_call = functools.partial(
        pl.kernel,
        mesh=plsc.VectorSubcoreMesh(core_axis_name="c", subcore_axis_name="s"),
        out_type=jax.ShapeDtypeStruct((_NW, _L), jnp.float32),
        compiler_params=pltpu.CompilerParams(needs_layout_passes=False),
        scratch_types=[
            pltpu.VMEM((_L,), jnp.float32),
            pltpu.VMEM((4 * _NPAD,), jnp.float32),
            pltpu.VMEM((4 * _NPAD,), jnp.float32),
            pltpu.VMEM((_L,), jnp.float32),
        ],
    )(_sc_kernel)
    sc_parts = sc_call(b1_flat, b2_flat)

    tri_tc, sumia, cov2 = pl.pallas_call(
        _tc_kernel,
        out_shape=[
            jax.ShapeDtypeStruct((1, 1), jnp.float32),
            jax.ShapeDtypeStruct((1, 1), jnp.float32),
            jax.ShapeDtypeStruct((1, _NPAD), jnp.float32),
        ],
        scratch_shapes=[
            pltpu.VMEM((8, _NPAD), jnp.float32),
            pltpu.VMEM((4 * _C, _C), jnp.float32),
        ],
    )(box1, b2c)

    tri = tri_tc[0, 0] + jnp.sum(sc_parts)
    b1_area = (box1[:, 2] - box1[:, 0] + 1.0) * (box1[:, 3] - box1[:, 1] + 1.0)
    cov1 = (sumia[0, 0] - tri) / b1_area
    return cov1, cov2[0, :_N]


# speedup vs baseline: 2.5935x; 1.8428x over previous
import jax
import jax.numpy as jnp
from jax.experimental import pallas as pl
from jax.experimental.pallas import tpu as pltpu

_N = 5000
_NPAD = 5120
_C = 256
_T = _NPAD // _C
_BIG = 1e9


def _oversegment_kernel(box1_ref, b2c_ref, cov1_ref, cov2_ref, cols_ref, rows_ref):
    b1x1 = box1_ref[0:1, 0:1]
    b1y1 = box1_ref[0:1, 1:2]
    b1x2 = box1_ref[0:1, 2:3]
    b1y2 = box1_ref[0:1, 3:4]

    x1c = b2c_ref[0:1, :]
    y1c = b2c_ref[1:2, :]
    x2c = b2c_ref[2:3, :]
    y2c = b2c_ref[3:4, :]

    ix1 = jnp.maximum(b1x1, x1c)
    iy1 = jnp.maximum(b1y1, y1c)
    ix2 = jnp.minimum(b1x2, x2c)
    iy2 = jnp.minimum(b1y2, y2c)
    ia = jnp.maximum(ix2 - ix1 + 1.0, 0.0) * jnp.maximum(iy2 - iy1 + 1.0, 0.0)
    valid = ia > 0.0
    cols_ref[0:1, :] = jnp.where(valid, ix1, _BIG)
    cols_ref[1:2, :] = jnp.where(valid, iy1, _BIG)
    cols_ref[2:3, :] = jnp.where(valid, ix2, -_BIG) + 1.0
    cols_ref[3:4, :] = jnp.where(valid, iy2, -_BIG) + 1.0

    b2area = (x2c - x1c + 1.0) * (y2c - y1c + 1.0)
    cov2_ref[...] = ia / b2area

    sum_ia = jnp.sum(ia)

    zvec = jnp.zeros((1, _C), jnp.float32)

    def outer(i, carry):
        a_up, a_diag = carry
        rs = pl.ds(i * _C, _C)
        rows_ref[0:_C, :] = jnp.broadcast_to(cols_ref[0:1, rs], (_C, _C)).T
        rows_ref[_C : 2 * _C, :] = jnp.broadcast_to(cols_ref[1:2, rs], (_C, _C)).T
        rows_ref[2 * _C : 3 * _C, :] = jnp.broadcast_to(cols_ref[2:3, rs], (_C, _C)).T
        rows_ref[3 * _C : 4 * _C, :] = jnp.broadcast_to(cols_ref[3:4, rs], (_C, _C)).T

        def inner(j, carry2):
            su, sd = carry2
            cs = pl.ds(j * _C, _C)
            cx1 = cols_ref[0:1, cs]
            cy1 = cols_ref[1:2, cs]
            cx2p = cols_ref[2:3, cs]
            cy2p = cols_ref[3:4, cs]
            t = jnp.zeros((1, _C), jnp.float32)
            for s in range(0, _C, 64):
                x_ext = jnp.maximum(rows_ref[2 * _C + s : 2 * _C + s + 64, :], cx2p) - jnp.maximum(
                    rows_ref[s : s + 64, :], cx1
                )
                y_ext = jnp.maximum(rows_ref[3 * _C + s : 3 * _C + s + 64, :], cy2p) - jnp.maximum(
                    rows_ref[_C + s : _C + s + 64, :], cy1
                )
                p = jnp.maximum(x_ext, 0.0) * jnp.maximum(y_ext, 0.0)
                t = t + jnp.sum(p, axis=0, keepdims=True)
            is_diag = j == i
            su = su + jnp.where(is_diag, zvec, t)
            sd = sd + jnp.where(is_diag, t, zvec)
            return su, sd

        return jax.lax.fori_loop(i, _T, inner, (a_up, a_diag))

    a_up, a_diag = jax.lax.fori_loop(0, _T, outer, (zvec, zvec))
    s_up = jnp.sum(a_up)
    s_diag = jnp.sum(a_diag)

    tri = s_up + (s_diag - sum_ia) * 0.5
    b1area = (b1x2 - b1x1 + 1.0) * (b1y2 - b1y1 + 1.0)
    cov1_ref[...] = (sum_ia - tri) / b1area


def kernel(box1, box2):
    pad = jnp.tile(
        jnp.array([[_BIG, _BIG, -_BIG, -_BIG]], dtype=jnp.float32),
        (_NPAD - _N, 1),
    )
    b2p = jnp.concatenate([box2.astype(jnp.float32), pad], axis=0)
    b2c = b2p.T

    cov1, cov2 = pl.pallas_call(
        _oversegment_kernel,
        out_shape=[
            jax.ShapeDtypeStruct((1, 1), jnp.float32),
            jax.ShapeDtypeStruct((1, _NPAD), jnp.float32),
        ],
        scratch_shapes=[
            pltpu.VMEM((8, _NPAD), jnp.float32),
            pltpu.VMEM((4 * _C, _C), jnp.float32),
        ],
    )(box1.astype(jnp.float32), b2c)

    return cov1.reshape(1), cov2[0, :_N]
